# Initial kernel scaffold; baseline (speedup 1.0000x reference)
import jax
import jax.numpy as jnp
from jax.experimental import pallas as pl


def kernel(hidden_states, gate_w, w_gate, w_up, w_down):
    raise NotImplementedError("write your pallas kernel here")


# speedup vs baseline: 1.2784x; 1.2784x over previous
import functools

import jax
import jax.numpy as jnp
from jax.experimental import pallas as pl
from jax.experimental.pallas import tpu as pltpu

LOAD_BALANCE_COEFF = 0.01


def _moe_body(x_ref, gw_ref, wg_ref, wu_ref, wd_ref,
              out_ref, aux_ref,
              cw_ref, facc_ref, pacc_ref,
              *, n_tb, n_e, n_tok, k):
    t = pl.program_id(0)
    e = pl.program_id(1)

    @pl.when(e == 0)
    def _router():
        x = x_ref[...]
        logits = jax.lax.dot_general(
            x, gw_ref[...], (((1,), (1,)), ((), ())),
            preferred_element_type=jnp.float32)
        m = jnp.max(logits, axis=1, keepdims=True)
        ex = jnp.exp(logits - m)
        p = ex / jnp.sum(ex, axis=1, keepdims=True)
        cols = jax.lax.broadcasted_iota(jnp.int32, p.shape, 1)
        big = jnp.int32(10 ** 9)
        m1 = jnp.max(p, axis=1, keepdims=True)
        i1 = jnp.min(jnp.where(p == m1, cols, big), axis=1, keepdims=True)
        mask1 = cols == i1
        p2 = jnp.where(mask1, -jnp.inf, p)
        m2 = jnp.max(p2, axis=1, keepdims=True)
        i2 = jnp.min(jnp.where(p2 == m2, cols, big), axis=1, keepdims=True)
        mask2 = cols == i2
        wsum = m1 + m2 + 1e-9
        cw_ref[...] = (jnp.where(mask1, m1 / wsum, 0.0)
                       + jnp.where(mask2, m2 / wsum, 0.0))

        @pl.when(t == 0)
        def _init_acc():
            facc_ref[...] = jnp.zeros_like(facc_ref)
            pacc_ref[...] = jnp.zeros_like(pacc_ref)
        hit = (mask1 | mask2).astype(jnp.float32)
        facc_ref[...] += jnp.sum(hit, axis=0, keepdims=True)
        pacc_ref[...] += jnp.sum(p, axis=0, keepdims=True)

    x = x_ref[...]
    g = jax.lax.dot_general(x, wg_ref[0], (((1,), (0,)), ((), ())),
                            preferred_element_type=jnp.float32)
    u = jax.lax.dot_general(x, wu_ref[0], (((1,), (0,)), ((), ())),
                            preferred_element_type=jnp.float32)
    act = (g / (1.0 + jnp.exp(-g))) * u
    y = jax.lax.dot_general(act, wd_ref[0], (((1,), (0,)), ((), ())),
                            preferred_element_type=jnp.float32)
    cw = cw_ref[...]
    ecols = jax.lax.broadcasted_iota(jnp.int32, cw.shape, 1)
    w_e = jnp.sum(jnp.where(ecols == e, cw, 0.0), axis=1, keepdims=True)

    @pl.when(e == 0)
    def _first():
        out_ref[...] = w_e * y

    @pl.when(e != 0)
    def _rest():
        out_ref[...] += w_e * y

    @pl.when((t == n_tb - 1) & (e == n_e - 1))
    def _aux():
        f_i = facc_ref[...] / (n_tok * k)
        p_i = pacc_ref[...] / n_tok
        aux_ref[0, 0] = LOAD_BALANCE_COEFF * n_e * jnp.sum(f_i * p_i)


def kernel(hidden_states, gate_w, w_gate, w_up, w_down):
    b, s, h = hidden_states.shape
    e, _, f = w_gate.shape
    t_tok = b * s
    x = hidden_states.reshape(t_tok, h)
    bt = min(256, t_tok)
    n_tb = t_tok // bt
    k = 2

    out, aux = pl.pallas_call(
        functools.partial(_moe_body, n_tb=n_tb, n_e=e, n_tok=t_tok, k=k),
        grid=(n_tb, e),
        in_specs=[
            pl.BlockSpec((bt, h), lambda t, i: (t, 0)),
            pl.BlockSpec((e, h), lambda t, i: (0, 0)),
            pl.BlockSpec((1, h, f), lambda t, i: (i, 0, 0)),
            pl.BlockSpec((1, h, f), lambda t, i: (i, 0, 0)),
            pl.BlockSpec((1, f, h), lambda t, i: (i, 0, 0)),
        ],
        out_specs=[
            pl.BlockSpec((bt, h), lambda t, i: (t, 0)),
            pl.BlockSpec(memory_space=pltpu.SMEM),
        ],
        out_shape=[
            jax.ShapeDtypeStruct((t_tok, h), jnp.float32),
            jax.ShapeDtypeStruct((1, 1), jnp.float32),
        ],
        scratch_shapes=[
            pltpu.VMEM((bt, e), jnp.float32),
            pltpu.VMEM((1, e), jnp.float32),
            pltpu.VMEM((1, e), jnp.float32),
        ],
    )(x, gate_w, w_gate, w_up, w_down)
    return out.reshape(b, s, h), aux[0, 0]
